# Initial kernel scaffold; baseline (speedup 1.0000x reference)
import jax
import jax.numpy as jnp
from jax.experimental import pallas as pl


def kernel(idx_tensor, table, attn_weight, attn_bias):
    raise NotImplementedError("write your pallas kernel here")


# speedup vs baseline: 1.7902x; 1.7902x over previous
import functools

import jax
import jax.numpy as jnp
from jax import lax
from jax.experimental import pallas as pl
from jax.experimental.pallas import tpu as pltpu
from jax.experimental.pallas import tpu_sc as plsc

PAD = 0
B, H, D = 4096, 200, 32
BH = B * H
HD = H * D


def _sc_gather(idx_flat, table):
    info = plsc.get_sparse_core_info()
    nw = info.num_cores * info.num_subcores
    per_w = BH // nw
    ch = 1600
    n_ch = per_w // ch
    mesh = plsc.VectorSubcoreMesh(core_axis_name="c", subcore_axis_name="s")

    @functools.partial(
        pl.kernel,
        mesh=mesh,
        out_type=jax.ShapeDtypeStruct((BH, D), jnp.float32),
        scratch_types=[
            pltpu.VMEM((ch,), jnp.int32),
            pltpu.VMEM((ch, D), jnp.float32),
            pltpu.SemaphoreType.DMA,
        ],
        compiler_params=pltpu.CompilerParams(use_tc_tiling_on_sc=False),
    )
    def k(idx_hbm, table_hbm, out_hbm, idx_v, rows_v, sem):
        wid = lax.axis_index("s") * info.num_cores + lax.axis_index("c")
        base = wid * per_w

        def body(i, carry):
            off = base + i * ch
            pltpu.sync_copy(idx_hbm.at[pl.ds(off, ch)], idx_v)
            pltpu.async_copy(table_hbm.at[idx_v], rows_v, sem).wait()
            pltpu.sync_copy(rows_v, out_hbm.at[pl.ds(off, ch)])
            return carry

        lax.fori_loop(0, n_ch, body, 0)

    return k(idx_flat, table)


def _tc_compute(idx, e2, w_sel, e_exp, r_sel):
    bc = 128
    grid = (B // bc,)

    def body(idx_ref, e_ref, ws_ref, ee_ref, r_ref, out_ref):
        idxb = idx_ref[...]
        e = e_ref[...]
        scores = jnp.dot(e, ws_ref[...], preferred_element_type=jnp.float32)
        valid = idxb != PAD
        has_real = jnp.any(valid, axis=1, keepdims=True)
        col = lax.broadcasted_iota(jnp.int32, (bc, H), 1)
        valid = valid | ((col == 0) & jnp.logical_not(has_real))
        scores = jnp.where(valid, scores, -jnp.inf)
        m = jnp.max(scores, axis=1, keepdims=True)
        p = jnp.exp(scores - m)
        z = jnp.sum(p, axis=1, keepdims=True)
        attn = p / z
        af = jnp.dot(attn, ee_ref[...], preferred_element_type=jnp.float32)
        out_ref[...] = jnp.dot(af * e, r_ref[...],
                               preferred_element_type=jnp.float32)

    return pl.pallas_call(
        body,
        grid=grid,
        in_specs=[
            pl.BlockSpec((bc, H), lambda i: (i, 0)),
            pl.BlockSpec((bc, HD), lambda i: (i, 0)),
            pl.BlockSpec((HD, H), lambda i: (0, 0)),
            pl.BlockSpec((H, HD), lambda i: (0, 0)),
            pl.BlockSpec((HD, D), lambda i: (0, 0)),
        ],
        out_specs=pl.BlockSpec((bc, D), lambda i: (i, 0)),
        out_shape=jax.ShapeDtypeStruct((B, D), jnp.float32),
    )(idx, e2, w_sel, e_exp, r_sel)


def _weight_mats(w):
    flat = jnp.arange(HD, dtype=jnp.int32)
    item = flat // D
    dim = flat % D
    items = jnp.arange(H, dtype=jnp.int32)
    dims = jnp.arange(D, dtype=jnp.int32)
    wt = jnp.tile(w, H)
    w_sel = jnp.where(item[:, None] == items[None, :], wt[:, None], 0.0)
    e_exp = (items[:, None] == item[None, :]).astype(jnp.float32)
    r_sel = (dim[:, None] == dims[None, :]).astype(jnp.float32)
    return w_sel, e_exp, r_sel


def kernel(idx_tensor, table, attn_weight, attn_bias):
    del attn_bias
    idx_flat = idx_tensor.reshape(-1)
    embs = _sc_gather(idx_flat, table)
    e2 = embs.reshape(B, HD)
    w_sel, e_exp, r_sel = _weight_mats(attn_weight.reshape(D))
    return _tc_compute(idx_tensor, e2, w_sel, e_exp, r_sel)
